# Initial kernel scaffold; baseline (speedup 1.0000x reference)
import jax
import jax.numpy as jnp
from jax.experimental import pallas as pl


def kernel(tree_x, edge_index, mutation_x, batch, params):
    raise NotImplementedError("write your pallas kernel here")


# speedup vs baseline: 6.2649x; 6.2649x over previous
import functools

import jax
import jax.numpy as jnp
from jax import lax
from jax.experimental import pallas as pl
from jax.experimental.pallas import tpu as pltpu
from jax.experimental.pallas import tpu_sc as plsc

N = 10000
E = 160000
B = 256
NACC = 10240
EPAD = 163840
PADROW = 10016
RB = 1000
NBLK = N // RB

_MESH = plsc.VectorSubcoreMesh(core_axis_name="c", subcore_axis_name="s")


def _leaky(x):
    return jnp.where(x >= 0, x, 0.01 * x)


def _ln(x, g, b):
    mu = jnp.mean(x, axis=-1, keepdims=True)
    var = jnp.mean((x - mu) ** 2, axis=-1, keepdims=True)
    return (x - mu) * lax.rsqrt(var + 1e-5) * g + b


@functools.partial(
    pl.kernel,
    out_type=jax.ShapeDtypeStruct((2 * NACC, 128), jnp.float32),
    mesh=_MESH,
    scratch_types=[
        pltpu.VMEM_SHARED((NACC, 128), jnp.float32),
        pltpu.VMEM((128,), jnp.int32),
        pltpu.VMEM((128, 128), jnp.float32),
    ],
    name="sc_deg",
)
def _sc_deg(dst_hbm, ones_hbm, zeros_hbm, out_hbm, acc, idx_v, ones_v):
    c = lax.axis_index("c")
    s = lax.axis_index("s")
    wid = c * 16 + s
    pltpu.sync_copy(ones_hbm, ones_v)
    pltpu.sync_copy(zeros_hbm, acc.at[pl.ds(s * 640, 640), :])
    plsc.subcore_barrier()
    ebase = pl.multiple_of(wid * 5120, 8)

    def body(k, _):
        pltpu.sync_copy(dst_hbm.at[pl.ds(ebase + k * 128, 128)], idx_v)
        pltpu.sync_copy(ones_v, acc.at[idx_v], add=True)
        return 0

    lax.fori_loop(0, 40, body, 0)
    plsc.subcore_barrier()
    pltpu.sync_copy(acc.at[pl.ds(s * 640, 640), :],
                    out_hbm.at[pl.ds(c * NACC + s * 640, 640), :])


@functools.partial(
    pl.kernel,
    out_type=jax.ShapeDtypeStruct((2 * NACC, 128), jnp.float32),
    mesh=_MESH,
    scratch_types=[
        pltpu.VMEM_SHARED((NACC, 128), jnp.float32),
        pltpu.VMEM((10240,), jnp.int32),
        pltpu.VMEM((128,), jnp.int32),
        pltpu.VMEM((128, 128), jnp.float32),
        pltpu.SemaphoreType.DMA,
    ],
    name="sc_seg128",
)
def _sc_seg128(tbl_hbm, src_hbm, dst_hbm, zeros_hbm, out_hbm,
               acc, src_v, idx_v, rows_v, sem):
    c = lax.axis_index("c")
    s = lax.axis_index("s")
    ebase = pl.multiple_of(s * 10240, 8)
    pltpu.sync_copy(src_hbm.at[pl.ds(ebase, 10240)], src_v)
    pltpu.sync_copy(zeros_hbm, acc.at[pl.ds(s * 640, 640), :])

    off = (c * N).astype(jnp.int32) if hasattr(c, "astype") else c * N

    def addoff(i, _):
        src_v[pl.ds(i * 16, 16)] = src_v[pl.ds(i * 16, 16)] + off
        return 0

    lax.fori_loop(0, 640, addoff, 0)
    plsc.subcore_barrier()

    def body(k, _):
        cp = pltpu.async_copy(tbl_hbm.at[src_v.at[pl.ds(k * 128, 128)]],
                              rows_v, sem)
        pltpu.sync_copy(dst_hbm.at[pl.ds(ebase + k * 128, 128)], idx_v)
        cp.wait()
        pltpu.sync_copy(rows_v, acc.at[idx_v], add=True)
        return 0

    lax.fori_loop(0, 80, body, 0)
    plsc.subcore_barrier()
    pltpu.sync_copy(acc.at[pl.ds(s * 640, 640), :],
                    out_hbm.at[pl.ds(c * NACC + s * 640, 640), :])


@functools.partial(
    pl.kernel,
    out_type=jax.ShapeDtypeStruct((2, B, 128), jnp.float32),
    mesh=_MESH,
    scratch_types=[
        pltpu.VMEM_SHARED((16, B, 128), jnp.float32),
        pltpu.VMEM((B, 128), jnp.float32),
        pltpu.VMEM((64, 128), jnp.float32),
        pltpu.VMEM((80,), jnp.int32),
        pltpu.VMEM((16, 128), jnp.float32),
        pltpu.VMEM((16, 128), jnp.float32),
    ],
    name="sc_segmax",
)
def _sc_segmax(x2_hbm, batch_hbm, neginf_hbm, out_hbm,
               smax, lout, xbuf, bbuf, abuf, pbuf):
    c = lax.axis_index("c")
    s = lax.axis_index("s")
    pltpu.sync_copy(neginf_hbm, lout)
    lo = jnp.minimum(s * 640, N - 640)

    def chunk(k, _):
        row0 = lo + k * 64
        pltpu.sync_copy(x2_hbm.at[pl.ds(c * N + row0, 64), :], xbuf)
        pltpu.sync_copy(batch_hbm.at[pl.ds(row0, 64)], bbuf.at[pl.ds(0, 64)])
        for j4 in range(4):
            bv = bbuf[pl.ds(j4 * 16, 16)]
            for l in range(16):
                bj = bv[l]
                j = j4 * 16 + l
                for f in range(8):
                    sl = pl.ds(f * 16, 16)
                    lout[bj, sl] = jnp.maximum(lout[bj, sl], xbuf[j, sl])
        return 0

    lax.fori_loop(0, 10, chunk, 0)
    pltpu.sync_copy(lout, smax.at[s])
    plsc.subcore_barrier()

    pltpu.sync_copy(smax.at[0, pl.ds(s * 16, 16), :], abuf)

    def merge(p, _):
        pltpu.sync_copy(smax.at[p, pl.ds(s * 16, 16), :], pbuf)
        for i in range(16):
            for f in range(8):
                sl = pl.ds(f * 16, 16)
                abuf[i, sl] = jnp.maximum(abuf[i, sl], pbuf[i, sl])
        return 0

    lax.fori_loop(1, 16, merge, 0)
    pltpu.sync_copy(abuf, out_hbm.at[c, pl.ds(s * 16, 16), :])


def _tc_prep_body(degp_ref, tx_ref, w1_ref, dinv_ref, h1_ref):
    deg = degp_ref[0, :, 0:1] + degp_ref[1, :, 0:1]
    dinv = lax.rsqrt(deg + 1.0)
    dinv_ref[...] = dinv
    x = tx_ref[...] * dinv
    h = jnp.dot(x, w1_ref[...], preferred_element_type=jnp.float32)
    h1_ref[0] = h[:, :128]
    h1_ref[1] = h[:, 128:]


def _tc_prep(degp, tree_x, w1):
    return pl.pallas_call(
        _tc_prep_body,
        grid=(NBLK,),
        in_specs=[
            pl.BlockSpec((2, RB, 128), lambda i: (0, i, 0)),
            pl.BlockSpec((RB, 8), lambda i: (i, 0)),
            pl.BlockSpec((8, 256), lambda i: (0, 0)),
        ],
        out_specs=[
            pl.BlockSpec((RB, 1), lambda i: (i, 0)),
            pl.BlockSpec((2, RB, 128), lambda i: (0, i, 0)),
        ],
        out_shape=[
            jax.ShapeDtypeStruct((N, 1), jnp.float32),
            jax.ShapeDtypeStruct((2, N, 128), jnp.float32),
        ],
        name="tc_prep",
    )(degp, tree_x, w1)


def _tc_b_body(s1p_ref, h1_ref, dinv_ref, b1_ref, g1_ref, be1_ref,
               w2_ref, h2_ref):
    dinv = dinv_ref[...]
    su = s1p_ref[...] + h1_ref[...]
    arr = jnp.concatenate([su[0], su[1]], axis=1)
    pre = dinv * arr + b1_ref[...]
    x1 = _leaky(_ln(pre, g1_ref[...], be1_ref[...]))
    h = jnp.dot(x1, w2_ref[...], preferred_element_type=jnp.float32) * dinv
    h2_ref[0] = h[:, :128]
    h2_ref[1] = h[:, 128:]


def _tc_b(s1p, h1, dinv, b1, g1, be1, w2):
    return pl.pallas_call(
        _tc_b_body,
        grid=(NBLK,),
        in_specs=[
            pl.BlockSpec((2, RB, 128), lambda i: (0, i, 0)),
            pl.BlockSpec((2, RB, 128), lambda i: (0, i, 0)),
            pl.BlockSpec((RB, 1), lambda i: (i, 0)),
            pl.BlockSpec((1, 256), lambda i: (0, 0)),
            pl.BlockSpec((1, 256), lambda i: (0, 0)),
            pl.BlockSpec((1, 256), lambda i: (0, 0)),
            pl.BlockSpec((256, 256), lambda i: (0, 0)),
        ],
        out_specs=pl.BlockSpec((2, RB, 128), lambda i: (0, i, 0)),
        out_shape=jax.ShapeDtypeStruct((2, N, 128), jnp.float32),
        name="tc_gcn1",
    )(s1p, h1, dinv, b1, g1, be1, w2)


def _tc_c_body(s2_ref, h2_ref, dinv_ref, b2_ref, g2_ref, be2_ref, x2_ref):
    su = s2_ref[...] + h2_ref[...]
    arr = jnp.concatenate([su[0], su[1]], axis=1)
    pre = dinv_ref[...] * arr + b2_ref[...]
    x2 = _leaky(_ln(pre, g2_ref[...], be2_ref[...]))
    x2_ref[0] = x2[:, :128]
    x2_ref[1] = x2[:, 128:]


def _tc_c(s2, h2, dinv, b2, g2, be2):
    return pl.pallas_call(
        _tc_c_body,
        grid=(NBLK,),
        in_specs=[
            pl.BlockSpec((2, RB, 128), lambda i: (0, i, 0)),
            pl.BlockSpec((2, RB, 128), lambda i: (0, i, 0)),
            pl.BlockSpec((RB, 1), lambda i: (i, 0)),
            pl.BlockSpec((1, 256), lambda i: (0, 0)),
            pl.BlockSpec((1, 256), lambda i: (0, 0)),
            pl.BlockSpec((1, 256), lambda i: (0, 0)),
        ],
        out_specs=pl.BlockSpec((2, RB, 128), lambda i: (0, i, 0)),
        out_shape=jax.ShapeDtypeStruct((2, N, 128), jnp.float32),
        name="tc_gcn2",
    )(s2, h2, dinv, b2, g2, be2)


def _tc_d_body(tm_ref, mx_ref, *refs):
    (tow, tob, tog, tobn,
     m0w, m0b, m1w, m1b, n1ag, n1ab, n1bg, n1bb,
     m2w, m2b, n2ag, n2ab, n2bg, n2bb,
     m3w, m3b, n3ag, n3ab, n3bg, n3bb, mow, mob,
     c0w, c0b, c1w, c1b, cn1g, cn1b,
     c2w, c2b, cn2g, cn2b,
     c3w, c3b, cn3g, cn3b,
     cow, cob, ow, ob, out_ref) = refs

    def dot(a, w, b):
        return jnp.dot(a, w[...], preferred_element_type=jnp.float32) + b[...]

    tm = tm_ref[...]
    t0 = jnp.concatenate([tm[0], tm[1]], axis=1)
    t0 = jnp.where(jnp.isfinite(t0), t0, 0.0)
    t = _leaky(_ln(dot(t0, tow, tob), tog[...], tobn[...]))

    m = mx_ref[...]
    m = dot(m, m0w, m0b)
    m = dot(m, m1w, m1b)
    m = _ln(m, n1ag[...], n1ab[...])
    m = _ln(m, n1bg[...], n1bb[...])
    m = dot(m, m2w, m2b)
    m = _ln(m, n2ag[...], n2ab[...])
    m = _ln(m, n2bg[...], n2bb[...])
    m = dot(m, m3w, m3b)
    m = _ln(m, n3ag[...], n3ab[...])
    m = _ln(m, n3bg[...], n3bb[...])
    m = dot(m, mow, mob)

    x = jnp.concatenate([t, m, t * m], axis=1)
    x = dot(x, c0w, c0b)
    x = dot(x, c1w, c1b)
    x = _leaky(_ln(x, cn1g[...], cn1b[...]))
    x = dot(x, c2w, c2b)
    x = _leaky(_ln(x, cn2g[...], cn2b[...]))
    x = dot(x, c3w, c3b)
    x = _leaky(_ln(x, cn3g[...], cn3b[...]))
    x = dot(x, cow, cob)
    out_ref[...] = dot(x, ow, ob)


def _tc_d(tm, mx, *ws):
    return pl.pallas_call(
        _tc_d_body,
        out_shape=jax.ShapeDtypeStruct((B, 1), jnp.float32),
        name="tc_head",
    )(tm, mx, *ws)


def kernel(tree_x, edge_index, mutation_x, batch, params):
    p = params
    f32 = jnp.float32
    src = edge_index[0].astype(jnp.int32)
    dst = edge_index[1].astype(jnp.int32)
    srcp = jnp.concatenate([src, jnp.zeros((EPAD - E,), jnp.int32)])
    dstp = jnp.concatenate([dst, jnp.full((EPAD - E,), PADROW, jnp.int32)])
    ones128 = jnp.ones((128, 128), f32)
    zeros128 = jnp.zeros((640, 128), f32)
    neginf = jnp.full((B, 128), -jnp.inf, f32)

    degp = _sc_deg(dstp, ones128, zeros128).reshape(2, NACC, 128)
    dinv, h1 = _tc_prep(degp, tree_x, p["gnn1_W"])

    s1p = _sc_seg128(h1.reshape(2 * N, 128), srcp, dstp,
                     zeros128).reshape(2, NACC, 128)
    h2 = _tc_b(s1p, h1, dinv,
               p["gnn1_b"].reshape(1, 256), p["gnn_norm1_g"].reshape(1, 256),
               p["gnn_norm1_b"].reshape(1, 256), p["gnn2_W"])

    s2p = _sc_seg128(h2.reshape(2 * N, 128), srcp, dstp,
                     zeros128).reshape(2, NACC, 128)
    x2 = _tc_c(s2p, h2, dinv, p["gnn2_b"].reshape(1, 256),
               p["gnn_norm2_g"].reshape(1, 256),
               p["gnn_norm2_b"].reshape(1, 256))

    arr = jnp.concatenate([x2[0], x2[1]], axis=1)
    t = jax.ops.segment_max(arr, batch, num_segments=B)
    tm = jnp.stack([t[:, :128], t[:, 128:]])

    def r2(name):
        return p[name].reshape(1, -1)

    out = _tc_d(
        tm, mutation_x,
        p["tree_out_W"], r2("tree_out_b"), r2("tree_out_norm_g"),
        r2("tree_out_norm_b"),
        p["m0_W"], r2("m0_b"), p["m1_W"], r2("m1_b"),
        r2("mn1a_g"), r2("mn1a_b"), r2("mn1b_g"), r2("mn1b_b"),
        p["m2_W"], r2("m2_b"), r2("mn2a_g"), r2("mn2a_b"),
        r2("mn2b_g"), r2("mn2b_b"),
        p["m3_W"], r2("m3_b"), r2("mn3a_g"), r2("mn3a_b"),
        r2("mn3b_g"), r2("mn3b_b"), p["m_out_W"], r2("m_out_b"),
        p["c0_W"], r2("c0_b"), p["c1_W"], r2("c1_b"),
        r2("cn1_g"), r2("cn1_b"),
        p["c2_W"], r2("c2_b"), r2("cn2_g"), r2("cn2_b"),
        p["c3_W"], r2("c3_b"), r2("cn3_g"), r2("cn3_b"),
        p["c_out_W"], r2("c_out_b"), p["out_W"], r2("out_b"),
    )
    return out
